# initial kernel scaffold (unmeasured)
import jax
import jax.numpy as jnp
from jax import lax
from jax.experimental import pallas as pl
from jax.experimental.pallas import tpu as pltpu

N_DEV = 4
B = 32
H = 16
D = 128
BS = 32
NB = 256
P_SHARD = 256
T_LOCAL = P_SHARD * BS
TB = 1024
NEG = -1e30
SCALE = D ** -0.5


def _attn_body(q_ref, k_ref, v_ref, w_ref, out_ref, acc, m_scr, l_scr):
    t = pl.program_id(0)
    nsteps = pl.num_programs(0)

    @pl.when(t == 0)
    def _init():
        acc[...] = jnp.zeros_like(acc)
        m_scr[...] = jnp.full_like(m_scr, NEG)
        l_scr[...] = jnp.zeros_like(l_scr)

    q_all = (q_ref[...] * SCALE).astype(jnp.bfloat16)
    w = w_ref[...]
    for h in range(H):
        rows = slice(h * B, (h + 1) * B)
        cols = slice(h * D, (h + 1) * D)
        q_h = q_all[:, cols]
        k_h = k_ref[:, cols].astype(jnp.bfloat16)
        v_h = v_ref[:, cols].astype(jnp.bfloat16)
        s = lax.dot_general(
            q_h, k_h, (((1,), (1,)), ((), ())),
            preferred_element_type=jnp.float32,
        )
        s = jnp.where(w > 0.0, s, NEG)
        m_prev = m_scr[rows, 0:1]
        m_cur = jnp.max(s, axis=1, keepdims=True)
        m_new = jnp.maximum(m_prev, m_cur)
        alpha = jnp.exp(m_prev - m_new)
        p = w * jnp.exp(s - m_new)
        l_new = l_scr[rows, 0:1] * alpha + jnp.sum(p, axis=1, keepdims=True)
        pv = lax.dot_general(
            p.astype(jnp.bfloat16), v_h, (((1,), (0,)), ((), ())),
            preferred_element_type=jnp.float32,
        )
        acc[rows, :] = acc[rows, :] * alpha + pv
        m_scr[rows, :] = jnp.broadcast_to(m_new, (B, D))
        l_scr[rows, :] = jnp.broadcast_to(l_new, (B, D))

    @pl.when(t == nsteps - 1)
    def _emit():
        for h in range(H):
            rows = slice(h * B, (h + 1) * B)
            out_ref[h, 0:B, :] = acc[rows, :]
            out_ref[h, B:2 * B, :] = m_scr[rows, :]
            out_ref[h, 2 * B:3 * B, :] = l_scr[rows, :]


def _ring_body(x_ref, out_ref, comm, send_sems, recv_sems):
    me = lax.axis_index("i")
    left = lax.rem(me + N_DEV - 1, N_DEV)
    right = lax.rem(me + 1, N_DEV)

    barrier = pltpu.get_barrier_semaphore()
    for nbr in (left, right):
        pl.semaphore_signal(
            barrier, inc=1,
            device_id=(nbr,), device_id_type=pl.DeviceIdType.MESH,
        )
    pl.semaphore_wait(barrier, 2)

    comm[0] = x_ref[...]
    for hop in range(N_DEV - 1):
        rdma = pltpu.make_async_remote_copy(
            src_ref=comm.at[hop],
            dst_ref=comm.at[hop + 1],
            send_sem=send_sems.at[hop],
            recv_sem=recv_sems.at[hop],
            device_id=(right,),
            device_id_type=pl.DeviceIdType.MESH,
        )
        rdma.start()
        rdma.wait()

    m_star = comm[0, :, B:2 * B, :]
    for s_i in range(1, N_DEV):
        m_star = jnp.maximum(m_star, comm[s_i, :, B:2 * B, :])
    num = jnp.zeros((H, B, D), jnp.float32)
    den = jnp.zeros((H, B, D), jnp.float32)
    for s_i in range(N_DEV):
        scale = jnp.exp(comm[s_i, :, B:2 * B, :] - m_star)
        num = num + scale * comm[s_i, :, 0:B, :]
        den = den + scale * comm[s_i, :, 2 * B:3 * B, :]
    out_ref[...] = num / den


def kernel(Q, K, V, bt, lens):
    me = lax.axis_index("i")

    Q2 = Q.reshape(B, H * D)
    K2 = K.reshape(T_LOCAL, H * D)
    V2 = V.reshape(T_LOCAL, H * D)

    start = me * P_SHARD
    slot_idx = jnp.arange(NB, dtype=jnp.int32)[None, :]
    valid = (
        (slot_idx < lens[:, None])
        & (bt >= start)
        & (bt < start + P_SHARD)
    )
    lid = bt - start
    page = jnp.arange(P_SHARD, dtype=jnp.int32)
    eq = (lid[:, :, None] == page[None, None, :]) & valid[:, :, None]
    counts = eq.sum(axis=1).astype(jnp.float32)
    w_tok = jnp.broadcast_to(
        counts[:, :, None], (B, P_SHARD, BS)
    ).reshape(B, T_LOCAL)

    nsteps = T_LOCAL // TB
    partial = pl.pallas_call(
        _attn_body,
        grid=(nsteps,),
        in_specs=[
            pl.BlockSpec((B, H * D), lambda t: (0, 0)),
            pl.BlockSpec((TB, H * D), lambda t: (t, 0)),
            pl.BlockSpec((TB, H * D), lambda t: (t, 0)),
            pl.BlockSpec((B, TB), lambda t: (0, t)),
        ],
        out_specs=pl.BlockSpec((H, 3 * B, D), lambda t: (0, 0, 0)),
        out_shape=jax.ShapeDtypeStruct((H, 3 * B, D), jnp.float32),
        scratch_shapes=[
            pltpu.VMEM((H * B, D), jnp.float32),
            pltpu.VMEM((H * B, D), jnp.float32),
            pltpu.VMEM((H * B, D), jnp.float32),
        ],
        compiler_params=pltpu.CompilerParams(
            dimension_semantics=("arbitrary",),
        ),
    )(Q2, K2, V2, w_tok)

    combined = pl.pallas_call(
        _ring_body,
        in_specs=[pl.BlockSpec(memory_space=pltpu.VMEM)],
        out_specs=pl.BlockSpec(memory_space=pltpu.VMEM),
        out_shape=jax.ShapeDtypeStruct((H, B, D), jnp.float32),
        scratch_shapes=[
            pltpu.VMEM((N_DEV, H, 3 * B, D), jnp.float32),
            pltpu.SemaphoreType.DMA((N_DEV - 1,)),
            pltpu.SemaphoreType.DMA((N_DEV - 1,)),
        ],
        compiler_params=pltpu.CompilerParams(collective_id=0),
    )(partial)

    return combined.transpose(1, 0, 2).reshape(B, 1, H, D)


# baseline (device time: 204747 ns/iter reference)
import jax
import jax.numpy as jnp
from jax import lax
from jax.experimental import pallas as pl
from jax.experimental.pallas import tpu as pltpu

N_DEV = 4
B = 32
H = 16
D = 128
BS = 32
NB = 256
P_SHARD = 256
T_LOCAL = P_SHARD * BS
TB = 1024
NEG = -1e30
SCALE = D ** -0.5


def _attn_body(q_ref, k_ref, v_ref, w_ref, out_ref, acc, m_scr, l_scr):
    t = pl.program_id(0)
    nsteps = pl.num_programs(0)

    @pl.when(t == 0)
    def _init():
        acc[...] = jnp.zeros_like(acc)
        m_scr[...] = jnp.full_like(m_scr, NEG)
        l_scr[...] = jnp.zeros_like(l_scr)

    q_all = (q_ref[...] * SCALE).astype(jnp.bfloat16)
    w = w_ref[...]
    for h in range(H):
        rows = slice(h * B, (h + 1) * B)
        cols = slice(h * D, (h + 1) * D)
        q_h = q_all[:, cols]
        k_h = k_ref[:, cols].astype(jnp.bfloat16)
        v_h = v_ref[:, cols].astype(jnp.bfloat16)
        s = lax.dot_general(
            q_h, k_h, (((1,), (1,)), ((), ())),
            preferred_element_type=jnp.float32,
        )
        s = jnp.where(w > 0.0, s, NEG)
        m_prev = m_scr[rows, 0:1]
        m_cur = jnp.max(s, axis=1, keepdims=True)
        m_new = jnp.maximum(m_prev, m_cur)
        alpha = jnp.exp(m_prev - m_new)
        p = w * jnp.exp(s - m_new)
        l_new = l_scr[rows, 0:1] * alpha + jnp.sum(p, axis=1, keepdims=True)
        pv = lax.dot_general(
            p.astype(jnp.bfloat16), v_h, (((1,), (0,)), ((), ())),
            preferred_element_type=jnp.float32,
        )
        acc[rows, :] = acc[rows, :] * alpha + pv
        m_scr[rows, :] = jnp.broadcast_to(m_new, (B, D))
        l_scr[rows, :] = jnp.broadcast_to(l_new, (B, D))

    @pl.when(t == nsteps - 1)
    def _emit():
        for h in range(H):
            rows = slice(h * B, (h + 1) * B)
            out_ref[h, 0:B, :] = acc[rows, :]
            out_ref[h, B:2 * B, :] = m_scr[rows, :]
            out_ref[h, 2 * B:3 * B, :] = l_scr[rows, :]


def _ring_body(x_ref, out_ref, comm, send_sems, recv_sems):
    me = lax.axis_index("i")
    left = lax.rem(me + N_DEV - 1, N_DEV)
    right = lax.rem(me + 1, N_DEV)

    barrier = pltpu.get_barrier_semaphore()
    for nbr in (left, right):
        pl.semaphore_signal(
            barrier, inc=1,
            device_id=(nbr,), device_id_type=pl.DeviceIdType.MESH,
        )
    pl.semaphore_wait(barrier, 2)

    comm[0] = x_ref[...]
    for hop in range(N_DEV - 1):
        rdma = pltpu.make_async_remote_copy(
            src_ref=comm.at[hop],
            dst_ref=comm.at[hop + 1],
            send_sem=send_sems.at[hop],
            recv_sem=recv_sems.at[hop],
            device_id=(right,),
            device_id_type=pl.DeviceIdType.MESH,
        )
        rdma.start()
        rdma.wait()

    m_star = comm[0, :, B:2 * B, :]
    for s_i in range(1, N_DEV):
        m_star = jnp.maximum(m_star, comm[s_i, :, B:2 * B, :])
    num = jnp.zeros((H, B, D), jnp.float32)
    den = jnp.zeros((H, B, D), jnp.float32)
    for s_i in range(N_DEV):
        scale = jnp.exp(comm[s_i, :, B:2 * B, :] - m_star)
        num = num + scale * comm[s_i, :, 0:B, :]
        den = den + scale * comm[s_i, :, 2 * B:3 * B, :]
    out_ref[...] = num / den


def kernel(Q, K, V, bt, lens):
    me = lax.axis_index("i")

    Q2 = Q.reshape(B, H * D)
    K2 = K.reshape(T_LOCAL, H * D)
    V2 = V.reshape(T_LOCAL, H * D)

    start = me * P_SHARD
    slot_idx = jnp.arange(NB, dtype=jnp.int32)[None, :]
    valid = (
        (slot_idx < lens[:, None])
        & (bt >= start)
        & (bt < start + P_SHARD)
    )
    lid = bt - start
    page = jnp.arange(P_SHARD, dtype=jnp.int32)
    eq = (lid[:, :, None] == page[None, None, :]) & valid[:, :, None]
    counts = eq.sum(axis=1).astype(jnp.float32)
    w_tok = jnp.broadcast_to(
        counts[:, :, None], (B, P_SHARD, BS)
    ).reshape(B, T_LOCAL)

    nsteps = T_LOCAL // TB
    partial = pl.pallas_call(
        _attn_body,
        grid=(nsteps,),
        in_specs=[
            pl.BlockSpec((B, H * D), lambda t: (0, 0)),
            pl.BlockSpec((TB, H * D), lambda t: (t, 0)),
            pl.BlockSpec((TB, H * D), lambda t: (t, 0)),
            pl.BlockSpec((B, TB), lambda t: (0, t)),
        ],
        out_specs=pl.BlockSpec((H, 3 * B, D), lambda t: (0, 0, 0)),
        out_shape=jax.ShapeDtypeStruct((H, 3 * B, D), jnp.float32),
        scratch_shapes=[
            pltpu.VMEM((H * B, D), jnp.float32),
            pltpu.VMEM((H * B, D), jnp.float32),
            pltpu.VMEM((H * B, D), jnp.float32),
        ],
        compiler_params=pltpu.CompilerParams(
            dimension_semantics=("arbitrary",),
            vmem_limit_bytes=64 * 1024 * 1024,
        ),
    )(Q2, K2, V2, w_tok)

    combined = pl.pallas_call(
        _ring_body,
        in_specs=[pl.BlockSpec(memory_space=pltpu.VMEM)],
        out_specs=pl.BlockSpec(memory_space=pltpu.VMEM),
        out_shape=jax.ShapeDtypeStruct((H, B, D), jnp.float32),
        scratch_shapes=[
            pltpu.VMEM((N_DEV, H, 3 * B, D), jnp.float32),
            pltpu.SemaphoreType.DMA((N_DEV - 1,)),
            pltpu.SemaphoreType.DMA((N_DEV - 1,)),
        ],
        compiler_params=pltpu.CompilerParams(collective_id=0),
    )(partial)

    return combined.transpose(1, 0, 2).reshape(B, 1, H, D)


# device time: 199772 ns/iter; 1.0249x vs baseline; 1.0249x over previous
import jax
import jax.numpy as jnp
from jax import lax
from jax.experimental import pallas as pl
from jax.experimental.pallas import tpu as pltpu

N_DEV = 4
B = 32
H = 16
D = 128
BS = 32
NB = 256
P_SHARD = 256
T_LOCAL = P_SHARD * BS
TB = 1024
PB = TB // BS
NEG = -1e30
SCALE = D ** -0.5


def _attn_body(q_ref, k_ref, v_ref, bt_ref, out_ref, acc, m_scr, l_scr, cnt_scr):
    t = pl.program_id(0)
    nsteps = pl.num_programs(0)

    @pl.when(t == 0)
    def _init():
        acc[...] = jnp.zeros_like(acc)
        m_scr[...] = jnp.full_like(m_scr, NEG)
        l_scr[...] = jnp.zeros_like(l_scr)
        bt_all = bt_ref[...]
        piota = lax.broadcasted_iota(jnp.int32, (B, P_SHARD), 1)
        cnt = jnp.zeros((B, P_SHARD), jnp.float32)
        for j in range(NB):
            bt_j = bt_all[:, j:j + 1]
            cnt = cnt + jnp.where(bt_j == piota, 1.0, 0.0)
        qi = lax.broadcasted_iota(jnp.int32, (B, B), 0)
        qj = lax.broadcasted_iota(jnp.int32, (B, B), 1)
        eye = (qi == qj).astype(jnp.bfloat16)
        cnt_t = lax.dot_general(
            cnt.astype(jnp.bfloat16), eye, (((0,), (0,)), ((), ())),
            preferred_element_type=jnp.float32,
        )
        cnt_scr[...] = cnt_t.astype(jnp.bfloat16)

    cnt_chunk = cnt_scr[pl.ds(t * PB, PB), :]
    sub_i = lax.broadcasted_iota(jnp.int32, (PB, TB), 0)
    tok_i = lax.broadcasted_iota(jnp.int32, (PB, TB), 1) // BS
    e_loc = (sub_i == tok_i).astype(jnp.bfloat16)
    w = lax.dot_general(
        cnt_chunk, e_loc, (((0,), (0,)), ((), ())),
        preferred_element_type=jnp.float32,
    )

    q_all = (q_ref[...] * SCALE).astype(jnp.bfloat16)
    k_bf = k_ref[...].astype(jnp.bfloat16)
    v_bf = v_ref[...].astype(jnp.bfloat16)
    for h in range(H):
        rows = slice(h * B, (h + 1) * B)
        cols = slice(h * D, (h + 1) * D)
        s = lax.dot_general(
            q_all[:, cols], k_bf[:, cols], (((1,), (1,)), ((), ())),
            preferred_element_type=jnp.float32,
        )
        s = jnp.where(w > 0.0, s, NEG)
        m_prev = m_scr[rows, 0:1]
        m_cur = jnp.max(s, axis=1, keepdims=True)
        m_new = jnp.maximum(m_prev, m_cur)
        alpha = jnp.exp(m_prev - m_new)
        p = w * jnp.exp(s - m_new)
        l_new = l_scr[rows, 0:1] * alpha + jnp.sum(p, axis=1, keepdims=True)
        pv = lax.dot_general(
            p.astype(jnp.bfloat16), v_bf[:, cols], (((1,), (0,)), ((), ())),
            preferred_element_type=jnp.float32,
        )
        acc[rows, :] = acc[rows, :] * alpha + pv
        m_scr[rows, :] = jnp.broadcast_to(m_new, (B, D))
        l_scr[rows, :] = jnp.broadcast_to(l_new, (B, D))

    @pl.when(t == nsteps - 1)
    def _emit():
        for h in range(H):
            rows = slice(h * B, (h + 1) * B)
            out_ref[h, 0:B, :] = acc[rows, :]
            out_ref[h, B:2 * B, :] = m_scr[rows, :]
            out_ref[h, 2 * B:3 * B, :] = l_scr[rows, :]


def _ring_body(x_ref, out_ref, comm, send_sems, recv_sems):
    me = lax.axis_index("i")
    left = lax.rem(me + N_DEV - 1, N_DEV)
    right = lax.rem(me + 1, N_DEV)

    barrier = pltpu.get_barrier_semaphore()
    for nbr in (left, right):
        pl.semaphore_signal(
            barrier, inc=1,
            device_id=(nbr,), device_id_type=pl.DeviceIdType.MESH,
        )
    pl.semaphore_wait(barrier, 2)

    comm[0] = x_ref[...]
    for hop in range(N_DEV - 1):
        rdma = pltpu.make_async_remote_copy(
            src_ref=comm.at[hop],
            dst_ref=comm.at[hop + 1],
            send_sem=send_sems.at[hop],
            recv_sem=recv_sems.at[hop],
            device_id=(right,),
            device_id_type=pl.DeviceIdType.MESH,
        )
        rdma.start()
        rdma.wait()

    m_star = comm[0, :, B:2 * B, :]
    for s_i in range(1, N_DEV):
        m_star = jnp.maximum(m_star, comm[s_i, :, B:2 * B, :])
    num = jnp.zeros((H, B, D), jnp.float32)
    den = jnp.zeros((H, B, D), jnp.float32)
    for s_i in range(N_DEV):
        scale = jnp.exp(comm[s_i, :, B:2 * B, :] - m_star)
        num = num + scale * comm[s_i, :, 0:B, :]
        den = den + scale * comm[s_i, :, 2 * B:3 * B, :]
    out_ref[...] = num / den


def kernel(Q, K, V, bt, lens):
    me = lax.axis_index("i")

    Q2 = Q.reshape(B, H * D)
    K2 = K.reshape(T_LOCAL, H * D)
    V2 = V.reshape(T_LOCAL, H * D)

    start = me * P_SHARD
    slot_idx = jnp.arange(NB, dtype=jnp.int32)[None, :]
    local = (
        (slot_idx < lens[:, None])
        & (bt >= start)
        & (bt < start + P_SHARD)
    )
    btm = jnp.where(local, bt - start, -1)

    nsteps = T_LOCAL // TB
    partial = pl.pallas_call(
        _attn_body,
        grid=(nsteps,),
        in_specs=[
            pl.BlockSpec((B, H * D), lambda t: (0, 0)),
            pl.BlockSpec((TB, H * D), lambda t: (t, 0)),
            pl.BlockSpec((TB, H * D), lambda t: (t, 0)),
            pl.BlockSpec((B, NB), lambda t: (0, 0)),
        ],
        out_specs=pl.BlockSpec((H, 3 * B, D), lambda t: (0, 0, 0)),
        out_shape=jax.ShapeDtypeStruct((H, 3 * B, D), jnp.float32),
        scratch_shapes=[
            pltpu.VMEM((H * B, D), jnp.float32),
            pltpu.VMEM((H * B, D), jnp.float32),
            pltpu.VMEM((H * B, D), jnp.float32),
            pltpu.VMEM((P_SHARD, B), jnp.bfloat16),
        ],
        compiler_params=pltpu.CompilerParams(
            dimension_semantics=("arbitrary",),
            vmem_limit_bytes=64 * 1024 * 1024,
        ),
    )(Q2, K2, V2, btm)

    combined = pl.pallas_call(
        _ring_body,
        in_specs=[pl.BlockSpec(memory_space=pltpu.VMEM)],
        out_specs=pl.BlockSpec(memory_space=pltpu.VMEM),
        out_shape=jax.ShapeDtypeStruct((H, B, D), jnp.float32),
        scratch_shapes=[
            pltpu.VMEM((N_DEV, H, 3 * B, D), jnp.float32),
            pltpu.SemaphoreType.DMA((N_DEV - 1,)),
            pltpu.SemaphoreType.DMA((N_DEV - 1,)),
        ],
        compiler_params=pltpu.CompilerParams(collective_id=0),
    )(partial)

    return combined.transpose(1, 0, 2).reshape(B, 1, H, D)


# device time: 119379 ns/iter; 1.7151x vs baseline; 1.6734x over previous
import jax
import jax.numpy as jnp
from jax import lax
from jax.experimental import pallas as pl
from jax.experimental.pallas import tpu as pltpu

N_DEV = 4
B = 32
H = 16
D = 128
BS = 32
NB = 256
P_SHARD = 256
T_LOCAL = P_SHARD * BS
TB = 2048
PB = TB // BS
NC = T_LOCAL // TB
NEG = -1e30
SCALE = D ** -0.5


def _attn_body(q_ref, k_hbm, v_hbm, bt_ref, out_ref,
               acc, m_scr, l_scr, w_scr, kbuf, vbuf, ksem, vsem):
    h = pl.program_id(0)
    c = pl.program_id(1)
    step = h * NC + c
    slot = lax.rem(step, 2)

    def _issue(ss, sl):
        hh = lax.div(ss, NC)
        cc = lax.rem(ss, NC)
        pltpu.make_async_copy(
            k_hbm.at[pl.ds(cc * PB, PB), :, hh, :], kbuf.at[sl], ksem.at[sl]
        ).start()
        pltpu.make_async_copy(
            v_hbm.at[pl.ds(cc * PB, PB), :, hh, :], vbuf.at[sl], vsem.at[sl]
        ).start()

    @pl.when(step == 0)
    def _warmup():
        _issue(step, slot)

    @pl.when(step + 1 < H * NC)
    def _prefetch():
        _issue(step + 1, lax.rem(step + 1, 2))

    @pl.when((h == 0) & (c == 0))
    def _weights():
        bt_all = bt_ref[...]
        piota = lax.broadcasted_iota(jnp.int32, (B, P_SHARD), 1)
        cnt = jnp.zeros((B, P_SHARD), jnp.float32)
        for j in range(NB):
            bt_j = bt_all[:, j:j + 1]
            cnt = cnt + jnp.where(bt_j == piota, 1.0, 0.0)
        qi = lax.broadcasted_iota(jnp.int32, (B, B), 0)
        qj = lax.broadcasted_iota(jnp.int32, (B, B), 1)
        eye = (qi == qj).astype(jnp.bfloat16)
        cnt_t = lax.dot_general(
            cnt.astype(jnp.bfloat16), eye, (((0,), (0,)), ((), ())),
            preferred_element_type=jnp.float32,
        ).astype(jnp.bfloat16)
        sub_i = lax.broadcasted_iota(jnp.int32, (PB, TB), 0)
        tok_i = lax.broadcasted_iota(jnp.int32, (PB, TB), 1) // BS
        e_loc = (sub_i == tok_i).astype(jnp.bfloat16)
        for cc in range(NC):
            cnt_chunk = cnt_t[cc * PB:(cc + 1) * PB, :]
            w_scr[cc] = lax.dot_general(
                cnt_chunk, e_loc, (((0,), (0,)), ((), ())),
                preferred_element_type=jnp.float32,
            )

    @pl.when(c == 0)
    def _init():
        acc[...] = jnp.zeros_like(acc)
        m_scr[...] = jnp.full_like(m_scr, NEG)
        l_scr[...] = jnp.zeros_like(l_scr)

    pltpu.make_async_copy(
        k_hbm.at[pl.ds(c * PB, PB), :, h, :], kbuf.at[slot], ksem.at[slot]
    ).wait()
    pltpu.make_async_copy(
        v_hbm.at[pl.ds(c * PB, PB), :, h, :], vbuf.at[slot], vsem.at[slot]
    ).wait()

    w = w_scr[c]
    q = (q_ref[...] * SCALE).astype(jnp.bfloat16)
    k = kbuf[slot].reshape(TB, D).astype(jnp.bfloat16)
    v = vbuf[slot].reshape(TB, D).astype(jnp.bfloat16)
    s = lax.dot_general(
        q, k, (((1,), (1,)), ((), ())),
        preferred_element_type=jnp.float32,
    )
    s = jnp.where(w > 0.0, s, NEG)
    m_prev = m_scr[:, 0:1]
    m_cur = jnp.max(s, axis=1, keepdims=True)
    m_new = jnp.maximum(m_prev, m_cur)
    alpha = jnp.exp(m_prev - m_new)
    p = w * jnp.exp(s - m_new)
    l_new = l_scr[:, 0:1] * alpha + jnp.sum(p, axis=1, keepdims=True)
    pv = lax.dot_general(
        p.astype(jnp.bfloat16), v, (((1,), (0,)), ((), ())),
        preferred_element_type=jnp.float32,
    )
    acc[...] = acc[...] * alpha + pv
    m_scr[...] = jnp.broadcast_to(m_new, (B, D))
    l_scr[...] = jnp.broadcast_to(l_new, (B, D))

    @pl.when(c == NC - 1)
    def _emit():
        out_ref[0:B, :] = acc[...]
        out_ref[B:2 * B, :] = m_scr[...]
        out_ref[2 * B:3 * B, :] = l_scr[...]


def _ring_body(x_ref, out_ref, comm, send_sems, recv_sems):
    me = lax.axis_index("i")
    left = lax.rem(me + N_DEV - 1, N_DEV)
    right = lax.rem(me + 1, N_DEV)

    barrier = pltpu.get_barrier_semaphore()
    for nbr in (left, right):
        pl.semaphore_signal(
            barrier, inc=1,
            device_id=(nbr,), device_id_type=pl.DeviceIdType.MESH,
        )
    pl.semaphore_wait(barrier, 2)

    comm[0] = x_ref[...]
    for hop in range(N_DEV - 1):
        rdma = pltpu.make_async_remote_copy(
            src_ref=comm.at[hop],
            dst_ref=comm.at[hop + 1],
            send_sem=send_sems.at[hop],
            recv_sem=recv_sems.at[hop],
            device_id=(right,),
            device_id_type=pl.DeviceIdType.MESH,
        )
        rdma.start()
        rdma.wait()

    m_star = comm[0, :, B:2 * B, :]
    for s_i in range(1, N_DEV):
        m_star = jnp.maximum(m_star, comm[s_i, :, B:2 * B, :])
    num = jnp.zeros((H, B, D), jnp.float32)
    den = jnp.zeros((H, B, D), jnp.float32)
    for s_i in range(N_DEV):
        scale = jnp.exp(comm[s_i, :, B:2 * B, :] - m_star)
        num = num + scale * comm[s_i, :, 0:B, :]
        den = den + scale * comm[s_i, :, 2 * B:3 * B, :]
    out_ref[...] = num / den


def kernel(Q, K, V, bt, lens):
    me = lax.axis_index("i")

    start = me * P_SHARD
    slot_idx = jnp.arange(NB, dtype=jnp.int32)[None, :]
    local = (
        (slot_idx < lens[:, None])
        & (bt >= start)
        & (bt < start + P_SHARD)
    )
    btm = jnp.where(local, bt - start, -1)

    Q2 = Q.reshape(B, H * D)
    partial = pl.pallas_call(
        _attn_body,
        grid=(H, NC),
        in_specs=[
            pl.BlockSpec((B, D), lambda h, c: (0, h)),
            pl.BlockSpec(memory_space=pltpu.MemorySpace.HBM),
            pl.BlockSpec(memory_space=pltpu.MemorySpace.HBM),
            pl.BlockSpec((B, NB), lambda h, c: (0, 0)),
        ],
        out_specs=pl.BlockSpec((None, 3 * B, D), lambda h, c: (h, 0, 0)),
        out_shape=jax.ShapeDtypeStruct((H, 3 * B, D), jnp.float32),
        scratch_shapes=[
            pltpu.VMEM((B, D), jnp.float32),
            pltpu.VMEM((B, D), jnp.float32),
            pltpu.VMEM((B, D), jnp.float32),
            pltpu.VMEM((NC, B, TB), jnp.float32),
            pltpu.VMEM((2, PB, BS, D), jnp.float32),
            pltpu.VMEM((2, PB, BS, D), jnp.float32),
            pltpu.SemaphoreType.DMA((2,)),
            pltpu.SemaphoreType.DMA((2,)),
        ],
        compiler_params=pltpu.CompilerParams(
            dimension_semantics=("arbitrary", "arbitrary"),
            vmem_limit_bytes=64 * 1024 * 1024,
        ),
    )(Q2, K, V, btm)

    combined = pl.pallas_call(
        _ring_body,
        in_specs=[pl.BlockSpec(memory_space=pltpu.VMEM)],
        out_specs=pl.BlockSpec(memory_space=pltpu.VMEM),
        out_shape=jax.ShapeDtypeStruct((H, B, D), jnp.float32),
        scratch_shapes=[
            pltpu.VMEM((N_DEV, H, 3 * B, D), jnp.float32),
            pltpu.SemaphoreType.DMA((N_DEV - 1,)),
            pltpu.SemaphoreType.DMA((N_DEV - 1,)),
        ],
        compiler_params=pltpu.CompilerParams(collective_id=0),
    )(partial)

    return combined.transpose(1, 0, 2).reshape(B, 1, H, D)


# device time: 64722 ns/iter; 3.1635x vs baseline; 1.8445x over previous
import jax
import jax.numpy as jnp
from jax import lax
from jax.experimental import pallas as pl
from jax.experimental.pallas import tpu as pltpu

N_DEV = 4
B = 32
H = 16
D = 128
BS = 32
NB = 256
P_SHARD = 256
T_LOCAL = P_SHARD * BS
TB = 8192
PB = TB // BS
NSPLIT = 1
NC = T_LOCAL // TB
NEG = -1e30
MINIT = -1e29
SCALE = D ** -0.5


def _attn_body(q_ref, k_hbm, v_hbm, bt_ref, out_ref,
               acc, m_scr, l_scr, w_scr, kbuf, vbuf, ksem, vsem):
    h = pl.program_id(0)
    c = pl.program_id(1)
    step = h * NC + c
    slot = lax.rem(step, 2)

    def _dmas(ss, sl):
        hh = lax.div(ss, NC)
        cc = lax.rem(ss, NC)
        sp = PB // NSPLIT
        out = []
        for i in range(NSPLIT):
            out.append(pltpu.make_async_copy(
                k_hbm.at[pl.ds(cc * PB + i * sp, sp), :, hh, :],
                kbuf.at[sl, pl.ds(i * sp, sp)], ksem.at[sl, i],
            ))
            out.append(pltpu.make_async_copy(
                v_hbm.at[pl.ds(cc * PB + i * sp, sp), :, hh, :],
                vbuf.at[sl, pl.ds(i * sp, sp)], vsem.at[sl, i],
            ))
        return out

    def _issue(ss, sl):
        for d in _dmas(ss, sl):
            d.start()

    @pl.when(step == 0)
    def _warmup():
        _issue(step, slot)

    @pl.when(step + 1 < H * NC)
    def _prefetch():
        _issue(step + 1, lax.rem(step + 1, 2))

    @pl.when((h == 0) & (c == 0))
    def _weights():
        bt_all = bt_ref[...]
        piota = lax.broadcasted_iota(jnp.int32, (B, P_SHARD), 1)
        cnt = jnp.zeros((B, P_SHARD), jnp.float32)
        for j in range(NB):
            bt_j = bt_all[:, j:j + 1]
            cnt = cnt + jnp.where(bt_j == piota, 1.0, 0.0)
        qi = lax.broadcasted_iota(jnp.int32, (B, B), 0)
        qj = lax.broadcasted_iota(jnp.int32, (B, B), 1)
        eye = (qi == qj).astype(jnp.bfloat16)
        cnt_t = lax.dot_general(
            cnt.astype(jnp.bfloat16), eye, (((0,), (0,)), ((), ())),
            preferred_element_type=jnp.float32,
        ).astype(jnp.bfloat16)
        sub_i = lax.broadcasted_iota(jnp.int32, (PB, TB), 0)
        tok_i = lax.broadcasted_iota(jnp.int32, (PB, TB), 1) // BS
        e_loc = (sub_i == tok_i).astype(jnp.bfloat16)
        for cc in range(NC):
            cnt_chunk = cnt_t[cc * PB:(cc + 1) * PB, :]
            wcc = lax.dot_general(
                cnt_chunk, e_loc, (((0,), (0,)), ((), ())),
                preferred_element_type=jnp.float32,
            )
            w_scr[cc] = jnp.where(wcc > 0.0, jnp.log(wcc), NEG)

    @pl.when(c == 0)
    def _init():
        acc[...] = jnp.zeros_like(acc)
        m_scr[...] = jnp.full_like(m_scr, MINIT)
        l_scr[...] = jnp.zeros_like(l_scr)

    for d in _dmas(step, slot):
        d.wait()

    w = w_scr[c]
    q = q_ref[...] * SCALE
    k = kbuf[slot].reshape(TB, D)
    v = vbuf[slot].reshape(TB, D)
    s = lax.dot_general(
        q, k, (((1,), (1,)), ((), ())),
        preferred_element_type=jnp.float32,
        precision=lax.Precision.DEFAULT,
    )
    s = s + w
    m_prev = m_scr[:, 0:1]
    m_cur = jnp.max(s, axis=1, keepdims=True)
    m_new = jnp.maximum(m_prev, m_cur)
    alpha = jnp.exp(m_prev - m_new)
    p = jnp.exp(s - m_new)
    l_new = l_scr[:, 0:1] * alpha + jnp.sum(p, axis=1, keepdims=True)
    pv = lax.dot_general(
        p, v, (((1,), (0,)), ((), ())),
        preferred_element_type=jnp.float32,
        precision=lax.Precision.DEFAULT,
    )
    acc[...] = acc[...] * alpha + pv
    m_scr[...] = jnp.broadcast_to(m_new, (B, D))
    l_scr[...] = jnp.broadcast_to(l_new, (B, D))

    @pl.when(c == NC - 1)
    def _emit():
        out_ref[0:B, :] = acc[...].astype(jnp.bfloat16)
        out_ref[B:2 * B, :] = m_scr[...].astype(jnp.bfloat16)
        out_ref[2 * B:3 * B, :] = l_scr[...].astype(jnp.bfloat16)


def _gather_body(x_ref, out_ref, comm, send_sems, recv_sems):
    me = lax.axis_index("i")

    barrier = pltpu.get_barrier_semaphore()
    for k in range(1, N_DEV):
        pl.semaphore_signal(
            barrier, inc=1,
            device_id=(lax.rem(me + k, N_DEV),),
            device_id_type=pl.DeviceIdType.MESH,
        )
    pl.semaphore_wait(barrier, N_DEV - 1)

    comm[0] = x_ref[...]
    rdmas = []
    for k in range(1, N_DEV):
        rdma = pltpu.make_async_remote_copy(
            src_ref=comm.at[0],
            dst_ref=comm.at[k],
            send_sem=send_sems.at[k - 1],
            recv_sem=recv_sems.at[k - 1],
            device_id=(lax.rem(me + k, N_DEV),),
            device_id_type=pl.DeviceIdType.MESH,
        )
        rdma.start()
        rdmas.append(rdma)
    for rdma in rdmas:
        rdma.wait()

    m_star = comm[0, :, B:2 * B, :].astype(jnp.float32)
    for s_i in range(1, N_DEV):
        m_star = jnp.maximum(
            m_star, comm[s_i, :, B:2 * B, :].astype(jnp.float32))
    num = jnp.zeros((H, B, D), jnp.float32)
    den = jnp.zeros((H, B, D), jnp.float32)
    for s_i in range(N_DEV):
        scale = jnp.exp(
            comm[s_i, :, B:2 * B, :].astype(jnp.float32) - m_star)
        num = num + scale * comm[s_i, :, 0:B, :].astype(jnp.float32)
        den = den + scale * comm[s_i, :, 2 * B:3 * B, :].astype(jnp.float32)
    out_ref[...] = num / den


def kernel(Q, K, V, bt, lens):
    me = lax.axis_index("i")

    start = me * P_SHARD
    slot_idx = jnp.arange(NB, dtype=jnp.int32)[None, :]
    local = (
        (slot_idx < lens[:, None])
        & (bt >= start)
        & (bt < start + P_SHARD)
    )
    btm = jnp.where(local, bt - start, -1)

    Q2 = Q.reshape(B, H * D)
    partial = pl.pallas_call(
        _attn_body,
        grid=(H, NC),
        in_specs=[
            pl.BlockSpec((B, D), lambda h, c: (0, h)),
            pl.BlockSpec(memory_space=pltpu.MemorySpace.HBM),
            pl.BlockSpec(memory_space=pltpu.MemorySpace.HBM),
            pl.BlockSpec((B, NB), lambda h, c: (0, 0)),
        ],
        out_specs=pl.BlockSpec((None, 3 * B, D), lambda h, c: (h, 0, 0)),
        out_shape=jax.ShapeDtypeStruct((H, 3 * B, D), jnp.bfloat16),
        scratch_shapes=[
            pltpu.VMEM((B, D), jnp.float32),
            pltpu.VMEM((B, D), jnp.float32),
            pltpu.VMEM((B, D), jnp.float32),
            pltpu.VMEM((NC, B, TB), jnp.float32),
            pltpu.VMEM((2, PB, BS, D), jnp.float32),
            pltpu.VMEM((2, PB, BS, D), jnp.float32),
            pltpu.SemaphoreType.DMA((2, NSPLIT)),
            pltpu.SemaphoreType.DMA((2, NSPLIT)),
        ],
        compiler_params=pltpu.CompilerParams(
            dimension_semantics=("arbitrary", "arbitrary"),
            vmem_limit_bytes=64 * 1024 * 1024,
        ),
    )(Q2, K, V, btm)

    combined = pl.pallas_call(
        _gather_body,
        in_specs=[pl.BlockSpec(memory_space=pltpu.VMEM)],
        out_specs=pl.BlockSpec(memory_space=pltpu.VMEM),
        out_shape=jax.ShapeDtypeStruct((H, B, D), jnp.float32),
        scratch_shapes=[
            pltpu.VMEM((N_DEV, H, 3 * B, D), jnp.bfloat16),
            pltpu.SemaphoreType.DMA((N_DEV - 1,)),
            pltpu.SemaphoreType.DMA((N_DEV - 1,)),
        ],
        compiler_params=pltpu.CompilerParams(collective_id=0),
    )(partial)

    return combined.transpose(1, 0, 2).reshape(B, 1, H, D)
